# Initial kernel scaffold; baseline (speedup 1.0000x reference)
import jax
import jax.numpy as jnp
from jax.experimental import pallas as pl


def kernel(edge_index, embeds):
    raise NotImplementedError("write your pallas kernel here")


# speedup vs baseline: 7.1289x; 7.1289x over previous
import functools

import jax
import jax.numpy as jnp
from jax import lax
from jax.experimental import pallas as pl
from jax.experimental.pallas import tpu as pltpu
from jax.experimental.pallas import tpu_sc as plsc

N = 10000
E = 320000
D = 128
DF = 144
SEED_NUM = 2000

NC = 2
NS = 16
NW = NC * NS
EPW = E // NW
W = 80
NWIN = EPW // W
NPAD = 10240
RPS = NPAD // NS

SR, SC_ = 128, 128
NSORT = SR * SC_


def _spmm144(row3, col3, src, zeros):
    mesh = plsc.VectorSubcoreMesh(core_axis_name="c", subcore_axis_name="s")

    @functools.partial(
        pl.kernel,
        mesh=mesh,
        compiler_params=pltpu.CompilerParams(use_tc_tiling_on_sc=False),
        out_type=jax.ShapeDtypeStruct((NC, NPAD, DF), jnp.float32),
        scratch_types=[
            pltpu.VMEM((NWIN, W), jnp.int32),
            pltpu.VMEM((NWIN, W), jnp.int32),
            pltpu.VMEM((W, DF), jnp.float32),
            pltpu.VMEM_SHARED((NPAD, DF), jnp.float32),
            pltpu.SemaphoreType.DMA,
        ],
    )
    def k(row_hbm, col_hbm, src_hbm, zero_hbm, out_hbm, rowv, colv, buf, acc,
          sem):
        c = lax.axis_index("c")
        s = lax.axis_index("s")
        wid = c * NS + s
        pltpu.sync_copy(zero_hbm.at[pl.ds(s * RPS, RPS)],
                        acc.at[pl.ds(s * RPS, RPS)])
        pltpu.sync_copy(row_hbm.at[wid], rowv)
        pltpu.sync_copy(col_hbm.at[wid], colv)
        plsc.subcore_barrier()

        @pl.loop(0, NWIN)
        def _(j):
            pltpu.async_copy(src_hbm.at[colv.at[j]], buf, sem).wait()
            pltpu.sync_copy(buf, acc.at[rowv.at[j]], add=True)

        plsc.subcore_barrier()
        pltpu.sync_copy(acc.at[pl.ds(s * RPS, RPS)],
                        out_hbm.at[c, pl.ds(s * RPS, RPS)])

    return k(row3, col3, src, zeros)


BR = 2000


def _mid_body(p_ref, x_ref, o_ref):
    ssum = p_ref[0] + p_ref[1]
    o_ref[:, :D] = ssum[:, :D] - x_ref[...]
    o_ref[:, D:] = ssum[:, D:]


def _mid(p1, x):
    return pl.pallas_call(
        _mid_body,
        grid=(N // BR,),
        in_specs=[
            pl.BlockSpec((2, BR, DF), lambda i: (0, i, 0)),
            pl.BlockSpec((BR, D), lambda i: (i, 0)),
        ],
        out_specs=pl.BlockSpec((BR, DF), lambda i: (i, 0)),
        out_shape=jax.ShapeDtypeStruct((N, DF), jnp.float32),
    )(p1, x)


def _final_body(p_ref, s2_ref, x_ref, n_ref, o_ref):
    ssum = p_ref[0] + p_ref[1]
    z = ssum[:, :D]
    w = ssum[:, D:D + 1]
    deg = s2_ref[:, D:D + 1]
    x = x_ref[...]
    numer = z - deg * x
    denom = w - deg + 1e-8
    sub = numer / denom
    sn = jnp.sqrt(jnp.sum(sub * sub, axis=1, keepdims=True))
    subn = sub / jnp.maximum(sn, 1e-12)
    xn = jnp.sqrt(jnp.sum(x * x, axis=1, keepdims=True))
    xu = x / jnp.maximum(xn, 1e-12)
    dot = jnp.sum(subn * xu, axis=1, keepdims=True)
    o_ref[...] = jnp.log(jax.nn.sigmoid(dot)) + n_ref[...]


def _final(p2, src2, x, noise_col):
    return pl.pallas_call(
        _final_body,
        grid=(N // BR,),
        in_specs=[
            pl.BlockSpec((2, BR, DF), lambda i: (0, i, 0)),
            pl.BlockSpec((BR, DF), lambda i: (i, 0)),
            pl.BlockSpec((BR, D), lambda i: (i, 0)),
            pl.BlockSpec((BR, 1), lambda i: (i, 0)),
        ],
        out_specs=pl.BlockSpec((BR, 1), lambda i: (i, 0)),
        out_shape=jax.ShapeDtypeStruct((N, 1), jnp.float32),
    )(p2, src2, x, noise_col)


def _sort_partner(a, d):
    if d >= SC_:
        dr = d // SC_
        lo = jnp.roll(a, -dr, axis=0)
        hi = jnp.roll(a, dr, axis=0)
        bit = (lax.broadcasted_iota(jnp.int32, (SR, SC_), 0) & dr) != 0
    else:
        lo = jnp.roll(a, -d, axis=1)
        hi = jnp.roll(a, d, axis=1)
        bit = (lax.broadcasted_iota(jnp.int32, (SR, SC_), 1) & d) != 0
    return jnp.where(bit, hi, lo)


def _sort_body(k_ref, oi_ref):
    K = k_ref[...]
    r = lax.broadcasted_iota(jnp.int32, (SR, SC_), 0)
    c = lax.broadcasted_iota(jnp.int32, (SR, SC_), 1)
    g = r * SC_ + c
    I = g
    for m in range(1, 15):
        blk = 1 << m
        flip = (g & blk) != 0
        for p in range(m - 1, -1, -1):
            d = 1 << p
            PK = _sort_partner(K, d)
            PI = _sort_partner(I, d)
            upper = (g & d) != 0
            better = (PK > K) | ((PK == K) & (PI < I))
            take = better ^ upper ^ flip
            K = jnp.where(take, PK, K)
            I = jnp.where(take, PI, I)
    oi_ref[...] = I


def _sort(keys):
    return pl.pallas_call(
        _sort_body,
        out_shape=jax.ShapeDtypeStruct((SR, SC_), jnp.int32),
    )(keys)


def kernel(edge_index, embeds):
    row3 = edge_index[0].reshape(NW, NWIN, W)
    col3 = edge_index[1].reshape(NW, NWIN, W)
    ones16 = jnp.ones((N, 16), jnp.float32)
    src1 = jnp.concatenate([embeds, ones16], axis=1)
    zeros = jnp.zeros((NPAD, DF), jnp.float32)

    p1 = _spmm144(row3, col3, src1, zeros)
    src2 = _mid(p1, embeds)
    p2 = _spmm144(row3, col3, src2, zeros)

    u = jax.random.uniform(jax.random.key(1), (N,), dtype=jnp.float32)
    u = jnp.where(u == 0, 1e-8, u)
    noise = -jnp.log(-jnp.log(u))

    scores = _final(p2, src2, embeds, noise.reshape(N, 1)).reshape(N)

    pad = jnp.full((NSORT - N,), -jnp.inf, dtype=jnp.float32)
    keys = jnp.concatenate([scores, pad]).reshape(SR, SC_)
    sidx = _sort(keys)
    seeds = sidx.reshape(-1)[:SEED_NUM]
    return scores, seeds
